# baseline (speedup 1.0000x reference)
import functools

import jax
import jax.numpy as jnp
from jax import lax
from jax.experimental import pallas as pl
from jax.experimental.pallas import tpu as pltpu

_DN = (((1,), (1,)), ((), ()))
_NC = 4


def _bi_rnn_kernel(
        xt_ref, xi_ref, xo_ref,
        wl_t_ref, wl_i_ref, wl_o_ref,
        bl_t_ref, bl_i_ref, bl_o_ref,
        wihf_t_ref, wihf_i_ref, wihf_o_ref,
        wihb_t_ref, wihb_i_ref, wihb_o_ref,
        whhf_t_ref, whhf_i_ref, whhf_o_ref,
        whhb_t_ref, whhb_i_ref, whhb_o_ref,
        bihf_t_ref, bihf_i_ref, bihf_o_ref,
        bhhf_t_ref, bhhf_i_ref, bhhf_o_ref,
        bihb_t_ref, bihb_i_ref, bihb_o_ref,
        bhhb_t_ref, bhhb_i_ref, bhhb_o_ref,
        out_ref,
        xst_scr, xsi_scr, xso_scr, gt_scr, gi_scr, go_scr,
        wl_scr, wih_scr, whh_scr, wihbf_scr, whht_scr, o_scr,
        x_sems, wl_sems, wih_sems, whh_sem, out_sem):
    T, Bh, two_h = o_scr.shape
    H = two_h // 2
    G = 4 * H
    D = xt_ref.shape[2]
    CT = T // _NC
    b0 = pl.program_id(0) * Bh

    xs_scrs = (xst_scr, xsi_scr, xso_scr)
    x_copies = [[pltpu.make_async_copy(
        x_hbm.at[pl.ds(b0, Bh), t, :],
        xs.at[pl.ds(t * Bh, Bh), :], x_sems.at[mi * _NC + t // CT])
        for t in range(T)]
        for mi, (x_hbm, xs) in enumerate(
            zip((xt_ref, xi_ref, xo_ref), xs_scrs))]
    wl_copies = [pltpu.make_async_copy(w, wl_scr.at[j], wl_sems.at[j])
                 for j, w in enumerate((wl_i_ref, wl_o_ref))]
    wih_copies = [
        (pltpu.make_async_copy(wf, wih_scr.at[j, 0], wih_sems.at[j]),
         pltpu.make_async_copy(wb, wih_scr.at[j, 1], wih_sems.at[j]))
        for j, (wf, wb) in enumerate(
            ((wihf_i_ref, wihb_i_ref), (wihf_o_ref, wihb_o_ref)))]
    whh_raw = (whhf_t_ref, whhb_t_ref, whhf_i_ref,
               whhb_i_ref, whhf_o_ref, whhb_o_ref)
    whh_copies = [pltpu.make_async_copy(w, whh_scr.at[s], whh_sem.at[0])
                  for s, w in enumerate(whh_raw)]

    for cp in x_copies[0]:
        cp.start()
    wl_copies[0].start()
    wih_copies[0][0].start()
    wih_copies[0][1].start()
    for cp in x_copies[1]:
        cp.start()
    wl_copies[1].start()
    wih_copies[1][0].start()
    wih_copies[1][1].start()
    for cp in x_copies[2]:
        cp.start()
    for cp in whh_copies:
        cp.start()

    g_scrs = (gt_scr, gi_scr, go_scr)
    bls = (bl_t_ref, bl_i_ref, bl_o_ref)
    bihfs = (bihf_t_ref, bihf_i_ref, bihf_o_ref)
    bhhfs = (bhhf_t_ref, bhhf_i_ref, bhhf_o_ref)
    bihbs = (bihb_t_ref, bihb_i_ref, bihb_o_ref)
    bhhbs = (bhhb_t_ref, bhhb_i_ref, bhhb_o_ref)

    def project(mi):
        bl2 = jnp.reshape(bls[mi][...], (1, D))
        bf2 = jnp.reshape(bihfs[mi][...] + bhhfs[mi][...], (1, G))
        bb2 = jnp.reshape(bihbs[mi][...] + bhhbs[mi][...], (1, G))
        if mi == 0:
            wl_val = wl_t_ref[...]
            wf_raw, wb_raw = wihf_t_ref[...], wihb_t_ref[...]
        else:
            wl_copies[mi - 1].wait()
            cf, cb = wih_copies[mi - 1]
            cf.wait()
            cb.wait()
            wl_val = wl_scr[mi - 1]
            wf_raw, wb_raw = wih_scr[mi - 1, 0], wih_scr[mi - 1, 1]
        wihbf_scr[mi, 0] = wf_raw.astype(jnp.bfloat16)
        wihbf_scr[mi, 1] = wb_raw.astype(jnp.bfloat16)
        xs, g_scr = xs_scrs[mi], g_scrs[mi]
        for c in range(_NC):
            for cp in x_copies[mi][c * CT:(c + 1) * CT]:
                cp.wait()
            r0, r1 = c * CT * Bh, (c + 1) * CT * Bh
            lin = lax.dot_general(xs[r0:r1, :], wl_val, _DN,
                                  preferred_element_type=jnp.float32)
            lin = (lin + bl2).astype(jnp.bfloat16)
            g_scr[r0:r1, 0:G] = lax.dot_general(
                lin, wihbf_scr[mi, 0], _DN,
                preferred_element_type=jnp.float32) + bf2
            g_scr[r0:r1, G:2 * G] = lax.dot_general(
                lin, wihbf_scr[mi, 1], _DN,
                preferred_element_type=jnp.float32) + bb2

    project(0)
    for cp in whh_copies:
        cp.wait()
    for s in range(6):
        whht_scr[s] = jnp.transpose(whh_scr[s][...], (1, 0)).astype(
            jnp.bfloat16)
    project(1)
    project(2)

    h = [jnp.zeros((Bh, H), jnp.float32) for _ in range(6)]
    c = [jnp.zeros((Bh, H), jnp.float32) for _ in range(6)]
    inv3 = 1.0 / 3.0

    out_copies = [pltpu.make_async_copy(
        o_scr.at[t], out_ref.at[pl.ds(b0, Bh), t, :], out_sem.at[0])
        for t in range(T)]

    for t in range(T):
        fwd_sum = jnp.zeros((Bh, H), jnp.float32)
        bwd_sum = jnp.zeros((Bh, H), jnp.float32)
        for m in range(3):
            for d in range(2):
                s = 2 * m + d
                tt = t if d == 0 else T - 1 - t
                gx = g_scrs[m][tt * Bh:(tt + 1) * Bh, d * G:(d + 1) * G]
                gsum = gx + jnp.dot(h[s].astype(jnp.bfloat16), whht_scr[s],
                                    preferred_element_type=jnp.float32)
                i_g = jax.nn.sigmoid(gsum[:, 0:H])
                f_g = jax.nn.sigmoid(gsum[:, H:2 * H])
                g_g = jnp.tanh(gsum[:, 2 * H:3 * H])
                o_g = jax.nn.sigmoid(gsum[:, 3 * H:4 * H])
                c[s] = f_g * c[s] + i_g * g_g
                h[s] = o_g * jnp.tanh(c[s])
                if d == 0:
                    fwd_sum = fwd_sum + h[s]
                else:
                    bwd_sum = bwd_sum + h[s]
        o_scr[t, :, 0:H] = fwd_sum * inv3
        o_scr[T - 1 - t, :, H:2 * H] = bwd_sum * inv3
        if t >= T // 2:
            out_copies[t].start()
            out_copies[T - 1 - t].start()
    for cp in out_copies:
        cp.wait()


_NCORES = 2


@functools.partial(jax.jit, static_argnames=())
def _bi_rnn_forward(*args33):
    x_text = args33[0]
    B, T, D = x_text.shape
    H = args33[15].shape[1]
    Bh = B // _NCORES

    def full(shape):
        zeros = (0,) * len(shape)
        return pl.BlockSpec(shape, lambda i, _z=zeros: _z)

    any_spec = pl.BlockSpec(memory_space=pl.ANY)
    vmem_auto = {3, 6, 7, 8, 9, 12} | set(range(21, 33))
    in_specs = [full(a.shape) if j in vmem_auto else any_spec
                for j, a in enumerate(args33)]

    return pl.pallas_call(
        _bi_rnn_kernel,
        out_shape=jax.ShapeDtypeStruct((B, T, 2 * H), jnp.float32),
        grid=(_NCORES,),
        in_specs=in_specs,
        out_specs=pl.BlockSpec(memory_space=pl.ANY),
        scratch_shapes=[
            pltpu.VMEM((T * Bh, D), jnp.float32),
            pltpu.VMEM((T * Bh, D), jnp.float32),
            pltpu.VMEM((T * Bh, D), jnp.float32),
            pltpu.VMEM((T * Bh, 8 * H), jnp.float32),
            pltpu.VMEM((T * Bh, 8 * H), jnp.float32),
            pltpu.VMEM((T * Bh, 8 * H), jnp.float32),
            pltpu.VMEM((2, D, D), jnp.float32),
            pltpu.VMEM((2, 2, 4 * H, D), jnp.float32),
            pltpu.VMEM((6, 4 * H, H), jnp.float32),
            pltpu.VMEM((3, 2, 4 * H, D), jnp.bfloat16),
            pltpu.VMEM((6, H, 4 * H), jnp.bfloat16),
            pltpu.VMEM((T, Bh, 2 * H), jnp.float32),
            pltpu.SemaphoreType.DMA((3 * _NC,)),
            pltpu.SemaphoreType.DMA((2,)),
            pltpu.SemaphoreType.DMA((2,)),
            pltpu.SemaphoreType.DMA((1,)),
            pltpu.SemaphoreType.DMA((1,)),
        ],
        compiler_params=pltpu.CompilerParams(
            dimension_semantics=("parallel",)),
    )(*args33)


def kernel(x_text, x_image, x_other,
           text__w_lin, text__b_lin,
           text__w_ih_f, text__w_hh_f, text__b_ih_f, text__b_hh_f,
           text__w_ih_b, text__w_hh_b, text__b_ih_b, text__b_hh_b,
           image__w_lin, image__b_lin,
           image__w_ih_f, image__w_hh_f, image__b_ih_f, image__b_hh_f,
           image__w_ih_b, image__w_hh_b, image__b_ih_b, image__b_hh_b,
           other__w_lin, other__b_lin,
           other__w_ih_f, other__w_hh_f, other__b_ih_f, other__b_hh_f,
           other__w_ih_b, other__w_hh_b, other__b_ih_b, other__b_hh_b):
    return _bi_rnn_forward(
        x_text, x_image, x_other,
        text__w_lin, image__w_lin, other__w_lin,
        text__b_lin, image__b_lin, other__b_lin,
        text__w_ih_f, image__w_ih_f, other__w_ih_f,
        text__w_ih_b, image__w_ih_b, other__w_ih_b,
        text__w_hh_f, image__w_hh_f, other__w_hh_f,
        text__w_hh_b, image__w_hh_b, other__w_hh_b,
        text__b_ih_f, image__b_ih_f, other__b_ih_f,
        text__b_hh_f, image__b_hh_f, other__b_hh_f,
        text__b_ih_b, image__b_ih_b, other__b_ih_b,
        text__b_hh_b, image__b_hh_b, other__b_hh_b)


# speedup vs baseline: 1.0246x; 1.0246x over previous
import functools

import jax
import jax.numpy as jnp
from jax import lax
from jax.experimental import pallas as pl
from jax.experimental.pallas import tpu as pltpu

_DN = (((1,), (1,)), ((), ()))


def _bi_rnn_kernel(
        xt_ref, xi_ref, xo_ref,
        wl_t_ref, wl_i_ref, wl_o_ref,
        bl_t_ref, bl_i_ref, bl_o_ref,
        wihf_t_ref, wihf_i_ref, wihf_o_ref,
        wihb_t_ref, wihb_i_ref, wihb_o_ref,
        whhf_t_ref, whhf_i_ref, whhf_o_ref,
        whhb_t_ref, whhb_i_ref, whhb_o_ref,
        bihf_t_ref, bihf_i_ref, bihf_o_ref,
        bhhf_t_ref, bhhf_i_ref, bhhf_o_ref,
        bihb_t_ref, bihb_i_ref, bihb_o_ref,
        bhhb_t_ref, bhhb_i_ref, bhhb_o_ref,
        out_ref,
        xst_scr, xsi_scr, xso_scr, gt_scr, gi_scr, go_scr, whht_scr,
        o_scr, dma_sems, out_sem):
    T, Bh, two_h = o_scr.shape
    H = two_h // 2
    G = 4 * H
    D = xt_ref.shape[2]
    b0 = pl.program_id(0) * Bh

    xs_scrs = (xst_scr, xsi_scr, xso_scr)
    copies = [[pltpu.make_async_copy(x_hbm.at[pl.ds(b0, Bh), t, :],
                                     xs.at[t], dma_sems.at[mi])
               for t in range(T)]
              for mi, (x_hbm, xs) in enumerate(
                  zip((xt_ref, xi_ref, xo_ref), xs_scrs))]
    for cs in copies:
        for cp in cs:
            cp.start()

    whh_raw = (whhf_t_ref, whhb_t_ref, whhf_i_ref,
               whhb_i_ref, whhf_o_ref, whhb_o_ref)
    for s in range(6):
        whht_scr[s] = jnp.transpose(
            whh_raw[s][...], (1, 0)).astype(jnp.bfloat16)

    def project(mi, wl_ref, bl_ref, wihf_ref, wihb_ref, bf2, bb2, g_scr):
        for cp in copies[mi]:
            cp.wait()
        lin = lax.dot_general(
            xs_scrs[mi][...].reshape(T * Bh, D), wl_ref[...], _DN,
            preferred_element_type=jnp.float32)
        lin = (lin + jnp.reshape(bl_ref[...], (1, D))).astype(jnp.bfloat16)
        g_scr[:, 0:G] = lax.dot_general(
            lin, wihf_ref[...].astype(jnp.bfloat16), _DN,
            preferred_element_type=jnp.float32) + bf2
        g_scr[:, G:2 * G] = lax.dot_general(
            lin, wihb_ref[...].astype(jnp.bfloat16), _DN,
            preferred_element_type=jnp.float32) + bb2

    mods = (
        (wl_t_ref, bl_t_ref, wihf_t_ref, wihb_t_ref,
         bihf_t_ref, bhhf_t_ref, bihb_t_ref, bhhb_t_ref, gt_scr),
        (wl_i_ref, bl_i_ref, wihf_i_ref, wihb_i_ref,
         bihf_i_ref, bhhf_i_ref, bihb_i_ref, bhhb_i_ref, gi_scr),
        (wl_o_ref, bl_o_ref, wihf_o_ref, wihb_o_ref,
         bihf_o_ref, bhhf_o_ref, bihb_o_ref, bhhb_o_ref, go_scr),
    )

    def proj_m(mi):
        (wl_ref, bl_ref, wihf_ref, wihb_ref,
         bihf_ref, bhhf_ref, bihb_ref, bhhb_ref, g_scr) = mods[mi]
        bf2 = jnp.reshape(bihf_ref[...] + bhhf_ref[...], (1, G))
        bb2 = jnp.reshape(bihb_ref[...] + bhhb_ref[...], (1, G))
        project(mi, wl_ref, bl_ref, wihf_ref, wihb_ref, bf2, bb2, g_scr)

    g_scrs = (gt_scr, gi_scr, go_scr)
    inv3 = 1.0 / 3.0

    out_copies = [pltpu.make_async_copy(
        o_scr.at[t], out_ref.at[pl.ds(b0, Bh), t, :], out_sem.at[0])
        for t in range(T)]

    def recur_m(m, last):
        g_scr = g_scrs[m]
        h = [jnp.zeros((Bh, H), jnp.float32) for _ in range(2)]
        c = [jnp.zeros((Bh, H), jnp.float32) for _ in range(2)]
        for t in range(T):
            for d in range(2):
                s = 2 * m + d
                tt = t if d == 0 else T - 1 - t
                gx = g_scr[tt * Bh:(tt + 1) * Bh, d * G:(d + 1) * G]
                gsum = gx + jnp.dot(h[d].astype(jnp.bfloat16), whht_scr[s],
                                    preferred_element_type=jnp.float32)
                i_g = jax.nn.sigmoid(gsum[:, 0:H])
                f_g = jax.nn.sigmoid(gsum[:, H:2 * H])
                g_g = jnp.tanh(gsum[:, 2 * H:3 * H])
                o_g = jax.nn.sigmoid(gsum[:, 3 * H:4 * H])
                c[d] = f_g * c[d] + i_g * g_g
                h[d] = o_g * jnp.tanh(c[d])
            hf3 = h[0] * inv3
            hb3 = h[1] * inv3
            if m == 0:
                o_scr[t, :, 0:H] = hf3
                o_scr[T - 1 - t, :, H:2 * H] = hb3
            else:
                o_scr[t, :, 0:H] = o_scr[t, :, 0:H] + hf3
                o_scr[T - 1 - t, :, H:2 * H] = (
                    o_scr[T - 1 - t, :, H:2 * H] + hb3)
            if last and t >= T // 2:
                out_copies[t].start()
                out_copies[T - 1 - t].start()

    proj_m(0)
    recur_m(0, False)
    proj_m(1)
    recur_m(1, False)
    proj_m(2)
    recur_m(2, True)
    for cp in out_copies:
        cp.wait()


_NCORES = 2


@functools.partial(jax.jit, static_argnames=())
def _bi_rnn_forward(*args33):
    x_text = args33[0]
    B, T, D = x_text.shape
    H = args33[15].shape[1]
    Bh = B // _NCORES

    def full(shape):
        zeros = (0,) * len(shape)
        return pl.BlockSpec(shape, lambda i, _z=zeros: _z)

    in_specs = [pl.BlockSpec(memory_space=pl.ANY) for _ in args33[:3]]
    in_specs += [full(a.shape) for a in args33[3:]]

    return pl.pallas_call(
        _bi_rnn_kernel,
        out_shape=jax.ShapeDtypeStruct((B, T, 2 * H), jnp.float32),
        grid=(_NCORES,),
        in_specs=in_specs,
        out_specs=pl.BlockSpec(memory_space=pl.ANY),
        scratch_shapes=[
            pltpu.VMEM((T, Bh, D), jnp.float32),
            pltpu.VMEM((T, Bh, D), jnp.float32),
            pltpu.VMEM((T, Bh, D), jnp.float32),
            pltpu.VMEM((T * Bh, 8 * H), jnp.float32),
            pltpu.VMEM((T * Bh, 8 * H), jnp.float32),
            pltpu.VMEM((T * Bh, 8 * H), jnp.float32),
            pltpu.VMEM((6, H, 4 * H), jnp.bfloat16),
            pltpu.VMEM((T, Bh, 2 * H), jnp.float32),
            pltpu.SemaphoreType.DMA((3,)),
            pltpu.SemaphoreType.DMA((1,)),
        ],
        compiler_params=pltpu.CompilerParams(
            dimension_semantics=("parallel",)),
    )(*args33)


def kernel(x_text, x_image, x_other,
           text__w_lin, text__b_lin,
           text__w_ih_f, text__w_hh_f, text__b_ih_f, text__b_hh_f,
           text__w_ih_b, text__w_hh_b, text__b_ih_b, text__b_hh_b,
           image__w_lin, image__b_lin,
           image__w_ih_f, image__w_hh_f, image__b_ih_f, image__b_hh_f,
           image__w_ih_b, image__w_hh_b, image__b_ih_b, image__b_hh_b,
           other__w_lin, other__b_lin,
           other__w_ih_f, other__w_hh_f, other__b_ih_f, other__b_hh_f,
           other__w_ih_b, other__w_hh_b, other__b_ih_b, other__b_hh_b):
    return _bi_rnn_forward(
        x_text, x_image, x_other,
        text__w_lin, image__w_lin, other__w_lin,
        text__b_lin, image__b_lin, other__b_lin,
        text__w_ih_f, image__w_ih_f, other__w_ih_f,
        text__w_ih_b, image__w_ih_b, other__w_ih_b,
        text__w_hh_f, image__w_hh_f, other__w_hh_f,
        text__w_hh_b, image__w_hh_b, other__w_hh_b,
        text__b_ih_f, image__b_ih_f, other__b_ih_f,
        text__b_hh_f, image__b_hh_f, other__b_hh_f,
        text__b_ih_b, image__b_ih_b, other__b_ih_b,
        text__b_hh_b, image__b_hh_b, other__b_hh_b)


# speedup vs baseline: 1.0506x; 1.0253x over previous
import functools

import numpy as np
import jax
import jax.numpy as jnp
from jax import lax
from jax.experimental import pallas as pl
from jax.experimental.pallas import tpu as pltpu

_DN = (((1,), (1,)), ((), ()))


def _bi_rnn_kernel(
        xt_ref, xi_ref, xo_ref,
        perm_ref,
        wl_t_ref, wl_i_ref, wl_o_ref,
        bl_t_ref, bl_i_ref, bl_o_ref,
        wihf_t_ref, wihf_i_ref, wihf_o_ref,
        wihb_t_ref, wihb_i_ref, wihb_o_ref,
        whhf_t_ref, whhf_i_ref, whhf_o_ref,
        whhb_t_ref, whhb_i_ref, whhb_o_ref,
        bihf_t_ref, bihf_i_ref, bihf_o_ref,
        bhhf_t_ref, bhhf_i_ref, bhhf_o_ref,
        bihb_t_ref, bihb_i_ref, bihb_o_ref,
        bhhb_t_ref, bhhb_i_ref, bhhb_o_ref,
        out_ref,
        gt_scr, gi_scr, go_scr, whht_scr, o_scr, out_sem):
    T, Bh, two_h = o_scr.shape
    H = two_h // 2
    G = 4 * H
    D = xt_ref.shape[2]
    b0 = pl.program_id(0) * Bh

    whh_raw = (whhf_t_ref, whhb_t_ref, whhf_i_ref,
               whhb_i_ref, whhf_o_ref, whhb_o_ref)
    for s in range(6):
        whht_scr[s] = jnp.transpose(
            whh_raw[s][...], (1, 0)).astype(jnp.bfloat16)

    x_refs = (xt_ref, xi_ref, xo_ref)

    def project(mi, wl_ref, bl_ref, wihf_ref, wihb_ref, bf2, bb2, g_scr):
        x2 = x_refs[mi][...].reshape(Bh * T, D)
        lin = lax.dot_general(
            x2, wl_ref[...], _DN,
            preferred_element_type=jnp.float32)
        lin = (lin + jnp.reshape(bl_ref[...], (1, D))).astype(jnp.bfloat16)
        lin = jnp.dot(perm_ref[...], lin,
                      preferred_element_type=jnp.float32).astype(jnp.bfloat16)
        g_scr[:, 0:G] = lax.dot_general(
            lin, wihf_ref[...].astype(jnp.bfloat16), _DN,
            preferred_element_type=jnp.float32) + bf2
        g_scr[:, G:2 * G] = lax.dot_general(
            lin, wihb_ref[...].astype(jnp.bfloat16), _DN,
            preferred_element_type=jnp.float32) + bb2

    mods = (
        (wl_t_ref, bl_t_ref, wihf_t_ref, wihb_t_ref,
         bihf_t_ref, bhhf_t_ref, bihb_t_ref, bhhb_t_ref, gt_scr),
        (wl_i_ref, bl_i_ref, wihf_i_ref, wihb_i_ref,
         bihf_i_ref, bhhf_i_ref, bihb_i_ref, bhhb_i_ref, gi_scr),
        (wl_o_ref, bl_o_ref, wihf_o_ref, wihb_o_ref,
         bihf_o_ref, bhhf_o_ref, bihb_o_ref, bhhb_o_ref, go_scr),
    )
    for mi, (wl_ref, bl_ref, wihf_ref, wihb_ref,
             bihf_ref, bhhf_ref, bihb_ref, bhhb_ref, g_scr) in enumerate(mods):
        bf2 = jnp.reshape(bihf_ref[...] + bhhf_ref[...], (1, G))
        bb2 = jnp.reshape(bihb_ref[...] + bhhb_ref[...], (1, G))
        project(mi, wl_ref, bl_ref, wihf_ref, wihb_ref, bf2, bb2, g_scr)

    g_scrs = (gt_scr, gi_scr, go_scr)

    h = [jnp.zeros((Bh, H), jnp.float32) for _ in range(6)]
    c = [jnp.zeros((Bh, H), jnp.float32) for _ in range(6)]
    inv3 = 1.0 / 3.0

    out_copies = [pltpu.make_async_copy(
        o_scr.at[t], out_ref.at[pl.ds(b0, Bh), t, :], out_sem.at[0])
        for t in range(T)]

    for t in range(T):
        fwd_sum = jnp.zeros((Bh, H), jnp.float32)
        bwd_sum = jnp.zeros((Bh, H), jnp.float32)
        for m in range(3):
            for d in range(2):
                s = 2 * m + d
                tt = t if d == 0 else T - 1 - t
                gx = g_scrs[m][tt * Bh:(tt + 1) * Bh, d * G:(d + 1) * G]
                gsum = gx + jnp.dot(h[s].astype(jnp.bfloat16), whht_scr[s],
                                    preferred_element_type=jnp.float32)
                i_g = jax.nn.sigmoid(gsum[:, 0:H])
                f_g = jax.nn.sigmoid(gsum[:, H:2 * H])
                g_g = jnp.tanh(gsum[:, 2 * H:3 * H])
                o_g = jax.nn.sigmoid(gsum[:, 3 * H:4 * H])
                c[s] = f_g * c[s] + i_g * g_g
                h[s] = o_g * jnp.tanh(c[s])
                if d == 0:
                    fwd_sum = fwd_sum + h[s]
                else:
                    bwd_sum = bwd_sum + h[s]
        o_scr[t, :, 0:H] = fwd_sum * inv3
        o_scr[T - 1 - t, :, H:2 * H] = bwd_sum * inv3
        if t >= T // 2:
            out_copies[t].start()
            out_copies[T - 1 - t].start()
    for cp in out_copies:
        cp.wait()


_NCORES = 2


@functools.partial(jax.jit, static_argnames=())
def _bi_rnn_forward(*args33):
    x_text = args33[0]
    B, T, D = x_text.shape
    H = args33[15].shape[1]
    Bh = B // _NCORES

    tb = T * Bh
    rows = np.arange(tb)
    perm = np.zeros((tb, tb), np.float32)
    perm[rows, (rows % Bh) * T + rows // Bh] = 1.0
    perm = jnp.asarray(perm.astype(jnp.bfloat16))

    def full(shape):
        zeros = (0,) * len(shape)
        return pl.BlockSpec(shape, lambda i, _z=zeros: _z)

    in_specs = [pl.BlockSpec((Bh, T, a.shape[2]), lambda i: (i, 0, 0))
                for a in args33[:3]]
    in_specs += [full(perm.shape)]
    in_specs += [full(a.shape) for a in args33[3:]]

    return pl.pallas_call(
        _bi_rnn_kernel,
        out_shape=jax.ShapeDtypeStruct((B, T, 2 * H), jnp.float32),
        grid=(_NCORES,),
        in_specs=in_specs,
        out_specs=pl.BlockSpec(memory_space=pl.ANY),
        scratch_shapes=[
            pltpu.VMEM((T * Bh, 8 * H), jnp.float32),
            pltpu.VMEM((T * Bh, 8 * H), jnp.float32),
            pltpu.VMEM((T * Bh, 8 * H), jnp.float32),
            pltpu.VMEM((6, H, 4 * H), jnp.bfloat16),
            pltpu.VMEM((T, Bh, 2 * H), jnp.float32),
            pltpu.SemaphoreType.DMA((1,)),
        ],
        compiler_params=pltpu.CompilerParams(
            dimension_semantics=("parallel",)),
    )(args33[0], args33[1], args33[2], perm, *args33[3:])


def kernel(x_text, x_image, x_other,
           text__w_lin, text__b_lin,
           text__w_ih_f, text__w_hh_f, text__b_ih_f, text__b_hh_f,
           text__w_ih_b, text__w_hh_b, text__b_ih_b, text__b_hh_b,
           image__w_lin, image__b_lin,
           image__w_ih_f, image__w_hh_f, image__b_ih_f, image__b_hh_f,
           image__w_ih_b, image__w_hh_b, image__b_ih_b, image__b_hh_b,
           other__w_lin, other__b_lin,
           other__w_ih_f, other__w_hh_f, other__b_ih_f, other__b_hh_f,
           other__w_ih_b, other__w_hh_b, other__b_ih_b, other__b_hh_b):
    return _bi_rnn_forward(
        x_text, x_image, x_other,
        text__w_lin, image__w_lin, other__w_lin,
        text__b_lin, image__b_lin, other__b_lin,
        text__w_ih_f, image__w_ih_f, other__w_ih_f,
        text__w_ih_b, image__w_ih_b, other__w_ih_b,
        text__w_hh_f, image__w_hh_f, other__w_hh_f,
        text__w_hh_b, image__w_hh_b, other__w_hh_b,
        text__b_ih_f, image__b_ih_f, other__b_ih_f,
        text__b_hh_f, image__b_hh_f, other__b_hh_f,
        text__b_ih_b, image__b_ih_b, other__b_ih_b,
        text__b_hh_b, image__b_hh_b, other__b_hh_b)


# speedup vs baseline: 1.2842x; 1.2223x over previous
import functools

import jax
import jax.numpy as jnp
from jax import lax
from jax.experimental import pallas as pl
from jax.experimental.pallas import tpu as pltpu

_DN = (((1,), (1,)), ((), ()))


def _bi_rnn_kernel(
        xt_ref, xi_ref, xo_ref,
        wl_t_ref, wl_i_ref, wl_o_ref,
        bl_t_ref, bl_i_ref, bl_o_ref,
        wihf_t_ref, wihf_i_ref, wihf_o_ref,
        wihb_t_ref, wihb_i_ref, wihb_o_ref,
        whhf_t_ref, whhf_i_ref, whhf_o_ref,
        whhb_t_ref, whhb_i_ref, whhb_o_ref,
        bihf_t_ref, bihf_i_ref, bihf_o_ref,
        bhhf_t_ref, bhhf_i_ref, bhhf_o_ref,
        bihb_t_ref, bihb_i_ref, bihb_o_ref,
        bhhb_t_ref, bhhb_i_ref, bhhb_o_ref,
        out_ref,
        xst_scr, xsi_scr, xso_scr, gt_scr, gi_scr, go_scr, whht_scr,
        o_scr, dma_sems, out_sem):
    T, Bh, two_h = o_scr.shape
    H = two_h // 2
    G = 4 * H
    D = xt_ref.shape[2]
    b0 = pl.program_id(0) * Bh

    xs_scrs = (xst_scr, xsi_scr, xso_scr)
    copies = [[pltpu.make_async_copy(x_hbm.at[pl.ds(b0, Bh), t, :],
                                     xs.at[t], dma_sems.at[mi])
               for t in range(T)]
              for mi, (x_hbm, xs) in enumerate(
                  zip((xt_ref, xi_ref, xo_ref), xs_scrs))]
    for cs in copies:
        for cp in cs:
            cp.start()

    whh_raw = (whhf_t_ref, whhb_t_ref, whhf_i_ref,
               whhb_i_ref, whhf_o_ref, whhb_o_ref)
    for s in range(6):
        whht_scr[s] = jnp.transpose(
            whh_raw[s][...], (1, 0)).astype(jnp.bfloat16)

    def project(mi, wl_ref, bl_ref, wihf_ref, wihb_ref, bf2, bb2, g_scr):
        for cp in copies[mi]:
            cp.wait()
        lin = lax.dot_general(
            xs_scrs[mi][...].reshape(T * Bh, D), wl_ref[...], _DN,
            preferred_element_type=jnp.float32)
        lin = (lin + jnp.reshape(bl_ref[...], (1, D))).astype(jnp.bfloat16)
        g_scr[:, 0:G] = lax.dot_general(
            lin, wihf_ref[...].astype(jnp.bfloat16), _DN,
            preferred_element_type=jnp.float32) + bf2
        g_scr[:, G:2 * G] = lax.dot_general(
            lin, wihb_ref[...].astype(jnp.bfloat16), _DN,
            preferred_element_type=jnp.float32) + bb2

    mods = (
        (wl_t_ref, bl_t_ref, wihf_t_ref, wihb_t_ref,
         bihf_t_ref, bhhf_t_ref, bihb_t_ref, bhhb_t_ref, gt_scr),
        (wl_i_ref, bl_i_ref, wihf_i_ref, wihb_i_ref,
         bihf_i_ref, bhhf_i_ref, bihb_i_ref, bhhb_i_ref, gi_scr),
        (wl_o_ref, bl_o_ref, wihf_o_ref, wihb_o_ref,
         bihf_o_ref, bhhf_o_ref, bihb_o_ref, bhhb_o_ref, go_scr),
    )
    for mi, (wl_ref, bl_ref, wihf_ref, wihb_ref,
             bihf_ref, bhhf_ref, bihb_ref, bhhb_ref, g_scr) in enumerate(mods):
        bf2 = jnp.reshape(bihf_ref[...] + bhhf_ref[...], (1, G))
        bb2 = jnp.reshape(bihb_ref[...] + bhhb_ref[...], (1, G))
        project(mi, wl_ref, bl_ref, wihf_ref, wihb_ref, bf2, bb2, g_scr)

    g_scrs = (gt_scr, gi_scr, go_scr)

    h = [jnp.zeros((Bh, H), jnp.float32) for _ in range(6)]
    c = [jnp.zeros((Bh, H), jnp.float32) for _ in range(6)]
    inv3 = 1.0 / 3.0

    out_copies = [pltpu.make_async_copy(
        o_scr.at[t], out_ref.at[pl.ds(b0, Bh), t, :], out_sem.at[0])
        for t in range(T)]

    for t in range(T):
        fwd_sum = jnp.zeros((Bh, H), jnp.float32)
        bwd_sum = jnp.zeros((Bh, H), jnp.float32)
        for m in range(3):
            for d in range(2):
                s = 2 * m + d
                tt = t if d == 0 else T - 1 - t
                gx = g_scrs[m][tt * Bh:(tt + 1) * Bh, d * G:(d + 1) * G]
                gsum = gx + jnp.dot(h[s].astype(jnp.bfloat16), whht_scr[s],
                                    preferred_element_type=jnp.float32)
                i_g = jax.nn.sigmoid(gsum[:, 0:H])
                f_g = jax.nn.sigmoid(gsum[:, H:2 * H])
                g_g = jnp.tanh(gsum[:, 2 * H:3 * H])
                o_g = jax.nn.sigmoid(gsum[:, 3 * H:4 * H])
                c[s] = f_g * c[s] + i_g * g_g
                h[s] = o_g * jnp.tanh(c[s])
                if d == 0:
                    fwd_sum = fwd_sum + h[s]
                else:
                    bwd_sum = bwd_sum + h[s]
        o_scr[t, :, 0:H] = fwd_sum * inv3
        o_scr[T - 1 - t, :, H:2 * H] = bwd_sum * inv3
        if t >= T // 2:
            out_copies[t].start()
            out_copies[T - 1 - t].start()
    for cp in out_copies:
        cp.wait()


_NCORES = 2


@functools.partial(jax.jit, static_argnames=())
def _bi_rnn_forward(*args33):
    x_text = args33[0]
    B, T, D = x_text.shape
    H = args33[15].shape[1]
    Bh = B // _NCORES

    def full(shape):
        zeros = (0,) * len(shape)
        return pl.BlockSpec(shape, lambda i, _z=zeros: _z)

    in_specs = [pl.BlockSpec(memory_space=pl.ANY) for _ in args33[:3]]
    in_specs += [full(a.shape) for a in args33[3:]]

    return pl.pallas_call(
        _bi_rnn_kernel,
        out_shape=jax.ShapeDtypeStruct((B, T, 2 * H), jnp.float32),
        grid=(_NCORES,),
        in_specs=in_specs,
        out_specs=pl.BlockSpec(memory_space=pl.ANY),
        scratch_shapes=[
            pltpu.VMEM((T, Bh, D), jnp.float32),
            pltpu.VMEM((T, Bh, D), jnp.float32),
            pltpu.VMEM((T, Bh, D), jnp.float32),
            pltpu.VMEM((T * Bh, 8 * H), jnp.float32),
            pltpu.VMEM((T * Bh, 8 * H), jnp.float32),
            pltpu.VMEM((T * Bh, 8 * H), jnp.float32),
            pltpu.VMEM((6, H, 4 * H), jnp.bfloat16),
            pltpu.VMEM((T, Bh, 2 * H), jnp.float32),
            pltpu.SemaphoreType.DMA((3,)),
            pltpu.SemaphoreType.DMA((1,)),
        ],
        compiler_params=pltpu.CompilerParams(
            dimension_semantics=("parallel",)),
    )(*args33)


def kernel(x_text, x_image, x_other,
           text__w_lin, text__b_lin,
           text__w_ih_f, text__w_hh_f, text__b_ih_f, text__b_hh_f,
           text__w_ih_b, text__w_hh_b, text__b_ih_b, text__b_hh_b,
           image__w_lin, image__b_lin,
           image__w_ih_f, image__w_hh_f, image__b_ih_f, image__b_hh_f,
           image__w_ih_b, image__w_hh_b, image__b_ih_b, image__b_hh_b,
           other__w_lin, other__b_lin,
           other__w_ih_f, other__w_hh_f, other__b_ih_f, other__b_hh_f,
           other__w_ih_b, other__w_hh_b, other__b_ih_b, other__b_hh_b):
    return _bi_rnn_forward(
        x_text, x_image, x_other,
        text__w_lin, image__w_lin, other__w_lin,
        text__b_lin, image__b_lin, other__b_lin,
        text__w_ih_f, image__w_ih_f, other__w_ih_f,
        text__w_ih_b, image__w_ih_b, other__w_ih_b,
        text__w_hh_f, image__w_hh_f, other__w_hh_f,
        text__w_hh_b, image__w_hh_b, other__w_hh_b,
        text__b_ih_f, image__b_ih_f, other__b_ih_f,
        text__b_hh_f, image__b_hh_f, other__b_hh_f,
        text__b_ih_b, image__b_ih_b, other__b_ih_b,
        text__b_hh_b, image__b_hh_b, other__b_hh_b)
